# baseline (speedup 1.0000x reference)
import dataclasses
import functools

import jax
import jax.numpy as jnp
from jax import lax
from jax.experimental import pallas as pl
from jax.experimental.pallas import tpu as pltpu
from jax.experimental.pallas import tpu_sc as plsc

VOCAB = 100000
DIM = 64
HOPS = 3
B, M, T = 1024, 50, 8

NT = HOPS + 1
PD = NT * DIM
LANES = 16
BLANES = 32
PDW = PD // 2
N_IDX = B * M * T
NC, NS = 2, 16
NW = NC * NS
ROWS_TOTAL = B * M
ROWS_PER_TILE = ROWS_TOTAL // NW
W = 80
N_CH = ROWS_PER_TILE // W
TH = T // 2

BB = 128


def _pool_body(idx_hbm, tab, dsts_hbm, out_hbm, idx_v, set_a, set_b, acc,
               dst_v, sem_a, sem_b):
    c = lax.axis_index("c")
    s = lax.axis_index("s")
    w = c * NS + s
    rbase = w * ROWS_PER_TILE

    for t in range(T):
        pltpu.sync_copy(
            idx_hbm.at[pl.ds(t * ROWS_TOTAL + rbase, ROWS_PER_TILE)],
            idx_v.at[pl.ds(t * ROWS_PER_TILE, ROWS_PER_TILE)])

    pltpu.sync_copy(dsts_hbm.at[w], dst_v)

    def isl(t, ch):
        return idx_v.at[pl.ds(t * ROWS_PER_TILE + ch * W, W)]

    def issue(set_ref, sem, ch, half):
        for t4 in range(TH):
            pltpu.async_copy(tab.at[isl(half * TH + t4, ch)],
                             set_ref.at[pl.ds(t4 * W, W)], sem)

    def drain(set_ref, sem, ch, half):
        for t4 in range(TH):
            pltpu.make_async_copy(tab.at[isl(half * TH + t4, ch)],
                                  set_ref.at[pl.ds(t4 * W, W)], sem).wait()

    issue(set_a, sem_a, 0, 0)
    issue(set_b, sem_b, 0, 1)

    @pl.loop(0, N_CH)
    def _ch(ch):
        drain(set_a, sem_a, ch, 0)

        @pl.loop(0, W)
        def _r0(r):
            for l in range(PDW // LANES):
                sl = pl.ds(l * LANES, LANES)
                v = (plsc.bitcast(set_a[r, sl], jnp.bfloat16)
                     + plsc.bitcast(set_a[W + r, sl], jnp.bfloat16)
                     + plsc.bitcast(set_a[2 * W + r, sl], jnp.bfloat16)
                     + plsc.bitcast(set_a[3 * W + r, sl], jnp.bfloat16))
                acc[r, sl] = plsc.bitcast(v, jnp.int32)

        @pl.when(ch + 1 < N_CH)
        def _():
            issue(set_a, sem_a, ch + 1, 0)

        drain(set_b, sem_b, ch, 1)

        @pl.loop(0, W)
        def _r1(r):
            for l in range(PDW // LANES):
                sl = pl.ds(l * LANES, LANES)
                v = (plsc.bitcast(acc[r, sl], jnp.bfloat16)
                     + plsc.bitcast(set_b[r, sl], jnp.bfloat16)
                     + plsc.bitcast(set_b[W + r, sl], jnp.bfloat16)
                     + plsc.bitcast(set_b[2 * W + r, sl], jnp.bfloat16)
                     + plsc.bitcast(set_b[3 * W + r, sl], jnp.bfloat16))
                acc[r, sl] = plsc.bitcast(v, jnp.int32)

        @pl.when(ch + 1 < N_CH)
        def _():
            issue(set_b, sem_b, ch + 1, 1)

        pltpu.sync_copy(acc, out_hbm.at[dst_v.at[ch]])


@jax.jit
def _pool(idx_t, tab, dsts):
    mesh = plsc.VectorSubcoreMesh(core_axis_name="c", subcore_axis_name="s")
    cp = pltpu.CompilerParams()
    if "needs_layout_passes" in pltpu.CompilerParams.__dataclass_fields__:
        cp = dataclasses.replace(cp, needs_layout_passes=False)
    k = pl.kernel(
        _pool_body,
        compiler_params=cp,
        out_type=jax.ShapeDtypeStruct((ROWS_TOTAL, PDW), jnp.int32),
        mesh=mesh,
        scratch_types=[
            pltpu.VMEM((T * ROWS_PER_TILE,), jnp.int32),
            pltpu.VMEM((TH * W, PDW), jnp.int32),
            pltpu.VMEM((TH * W, PDW), jnp.int32),
            pltpu.VMEM((W, PDW), jnp.int32),
            pltpu.VMEM((N_CH, W), jnp.int32),
            pltpu.SemaphoreType.DMA,
            pltpu.SemaphoreType.DMA,
        ],
    )
    return k(idx_t, tab, dsts)


def _hops_body(m_ref, u_ref):
    u = jnp.zeros((BB, DIM), jnp.float32)
    for hop in range(HOPS):
        m_a = m_ref[:, :, hop * DIM:(hop + 1) * DIM].astype(jnp.float32)
        logits = jnp.sum(m_a * u[None, :, :], axis=-1)
        e = jnp.exp(logits)
        p = e / jnp.sum(e, axis=0, keepdims=True)
        m_c = m_ref[:, :, (hop + 1) * DIM:(hop + 2) * DIM].astype(jnp.float32)
        u = u + jnp.sum(m_c * p[:, :, None], axis=0)
    u_ref[...] = u


def _hops(m):
    return pl.pallas_call(
        _hops_body,
        grid=(B // BB,),
        in_specs=[pl.BlockSpec((M, BB, PD), lambda i: (0, i, 0))],
        out_specs=pl.BlockSpec((BB, DIM), lambda i: (i, 0)),
        out_shape=jax.ShapeDtypeStruct((B, DIM), jnp.float32),
    )(m)


def kernel(story, C0, C1, C2, C3):
    idx_t = story.transpose(2, 0, 1).reshape(N_IDX)
    tab_bf = jnp.concatenate([C0, C1, C2, C3], axis=1).astype(jnp.bfloat16)
    tab = jax.lax.bitcast_convert_type(
        tab_bf.reshape(VOCAB, PDW, 2), jnp.int32)
    r = jnp.arange(ROWS_TOTAL, dtype=jnp.int32)
    dsts = ((r % M) * B + r // M).reshape(NW, N_CH, W)
    pooled = _pool(idx_t, tab, dsts)
    m_bf = jax.lax.bitcast_convert_type(pooled, jnp.bfloat16)
    return _hops(m_bf.reshape(M, B, PD))


# speedup vs baseline: 1.7464x; 1.7464x over previous
import functools

import jax
import jax.numpy as jnp
from jax import lax
from jax.experimental import pallas as pl
from jax.experimental.pallas import tpu as pltpu
from jax.experimental.pallas import tpu_sc as plsc

VOCAB = 100000
DIM = 64
HOPS = 3
B, M, T = 1024, 50, 8

NT = HOPS + 1
PD = NT * DIM
LANES = 16
N_IDX = B * M * T
NC, NS = 2, 16
NW = NC * NS
ROWS_TOTAL = B * M
ROWS_PER_TILE = ROWS_TOTAL // NW
W = 40
N_CH = ROWS_PER_TILE // W
TH = T // 2

BB = 128


def _pool_body(idx_hbm, tab, dsts_hbm, out_hbm, idx_v, set_a, set_b, acc,
               dst_v, sem_a, sem_b):
    c = lax.axis_index("c")
    s = lax.axis_index("s")
    w = c * NS + s
    rbase = w * ROWS_PER_TILE

    for t in range(T):
        pltpu.sync_copy(
            idx_hbm.at[pl.ds(t * ROWS_TOTAL + rbase, ROWS_PER_TILE)],
            idx_v.at[pl.ds(t * ROWS_PER_TILE, ROWS_PER_TILE)])

    pltpu.sync_copy(dsts_hbm.at[w], dst_v)

    def isl(t, ch):
        return idx_v.at[pl.ds(t * ROWS_PER_TILE + ch * W, W)]

    def issue(set_ref, sem, ch, half):
        for t4 in range(TH):
            pltpu.async_copy(tab.at[isl(half * TH + t4, ch)],
                             set_ref.at[pl.ds(t4 * W, W)], sem)

    def drain(set_ref, sem, ch, half):
        for t4 in range(TH):
            pltpu.make_async_copy(tab.at[isl(half * TH + t4, ch)],
                                  set_ref.at[pl.ds(t4 * W, W)], sem).wait()

    issue(set_a, sem_a, 0, 0)
    issue(set_b, sem_b, 0, 1)

    @pl.loop(0, N_CH)
    def _ch(ch):
        drain(set_a, sem_a, ch, 0)

        @pl.loop(0, W)
        def _r0(r):
            for l in range(PD // LANES):
                sl = pl.ds(l * LANES, LANES)
                acc[r, sl] = (set_a[r, sl] + set_a[W + r, sl]
                              + set_a[2 * W + r, sl] + set_a[3 * W + r, sl])

        @pl.when(ch + 1 < N_CH)
        def _():
            issue(set_a, sem_a, ch + 1, 0)

        drain(set_b, sem_b, ch, 1)

        @pl.loop(0, W)
        def _r1(r):
            for l in range(PD // LANES):
                sl = pl.ds(l * LANES, LANES)
                acc[r, sl] = (acc[r, sl]
                              + set_b[r, sl] + set_b[W + r, sl]
                              + set_b[2 * W + r, sl] + set_b[3 * W + r, sl])

        @pl.when(ch + 1 < N_CH)
        def _():
            issue(set_b, sem_b, ch + 1, 1)

        pltpu.sync_copy(acc, out_hbm.at[dst_v.at[ch]])


@jax.jit
def _pool(idx_t, tab, dsts):
    mesh = plsc.VectorSubcoreMesh(core_axis_name="c", subcore_axis_name="s")
    k = pl.kernel(
        _pool_body,
        out_type=jax.ShapeDtypeStruct((ROWS_TOTAL, PD), jnp.float32),
        mesh=mesh,
        scratch_types=[
            pltpu.VMEM((T * ROWS_PER_TILE,), jnp.int32),
            pltpu.VMEM((TH * W, PD), jnp.float32),
            pltpu.VMEM((TH * W, PD), jnp.float32),
            pltpu.VMEM((W, PD), jnp.float32),
            pltpu.VMEM((N_CH, W), jnp.int32),
            pltpu.SemaphoreType.DMA,
            pltpu.SemaphoreType.DMA,
        ],
    )
    return k(idx_t, tab, dsts)


def _hops_body(m_ref, u_ref):
    u = jnp.zeros((BB, DIM), jnp.float32)
    for hop in range(HOPS):
        m_a = m_ref[:, :, hop * DIM:(hop + 1) * DIM]
        logits = jnp.sum(m_a * u[None, :, :], axis=-1)
        e = jnp.exp(logits)
        p = e / jnp.sum(e, axis=0, keepdims=True)
        m_c = m_ref[:, :, (hop + 1) * DIM:(hop + 2) * DIM]
        u = u + jnp.sum(m_c * p[:, :, None], axis=0)
    u_ref[...] = u


def _hops(m):
    return pl.pallas_call(
        _hops_body,
        grid=(B // BB,),
        in_specs=[pl.BlockSpec((M, BB, PD), lambda i: (0, i, 0))],
        out_specs=pl.BlockSpec((BB, DIM), lambda i: (i, 0)),
        out_shape=jax.ShapeDtypeStruct((B, DIM), jnp.float32),
    )(m)


def kernel(story, C0, C1, C2, C3):
    idx_t = story.transpose(2, 0, 1).reshape(N_IDX)
    tab = jnp.concatenate([C0, C1, C2, C3], axis=1)
    r = jnp.arange(ROWS_TOTAL, dtype=jnp.int32)
    dsts = ((r % M) * B + r // M).reshape(NW, N_CH, W)
    pooled = _pool(idx_t, tab, dsts)
    return _hops(pooled.reshape(M, B, PD))
